# baseline (speedup 1.0000x reference)
import functools

import jax
import jax.numpy as jnp
from jax import lax
from jax.experimental import pallas as pl
from jax.experimental.pallas import tpu as pltpu
from jax.experimental.pallas import tpu_sc as plsc

HEADS = 4
PE_DIM = 64
NUM_CORES = 2
NUM_SUBCORES = 16
NUM_WORKERS = NUM_CORES * NUM_SUBCORES


def _sc_gather_rows(pjpe, jidx, pidx, rows_pad, rows_per_worker):
  mesh = plsc.VectorSubcoreMesh(
      core_axis_name="c",
      subcore_axis_name="s",
      num_cores=NUM_CORES,
      num_subcores=NUM_SUBCORES,
  )

  @functools.partial(
      pl.kernel,
      mesh=mesh,
      compiler_params=pltpu.CompilerParams(use_tc_tiling_on_sc=False),
      out_type=jax.ShapeDtypeStruct((2, rows_pad, PE_DIM), jnp.float32),
      scratch_types=[
          pltpu.VMEM((rows_per_worker,), jnp.int32),
          pltpu.VMEM((rows_per_worker, PE_DIM), jnp.float32),
          pltpu.SemaphoreType.DMA,
      ],
  )
  def gather_kernel(table_hbm, jidx_hbm, pidx_hbm, out_hbm, idx_v, rows_v, sem):
    wid = lax.axis_index("s") * NUM_CORES + lax.axis_index("c")
    base = wid * rows_per_worker
    pltpu.sync_copy(jidx_hbm.at[pl.ds(base, rows_per_worker)], idx_v)
    pltpu.async_copy(table_hbm.at[idx_v], rows_v, sem).wait()
    pltpu.sync_copy(rows_v, out_hbm.at[0].at[pl.ds(base, rows_per_worker)])
    pltpu.sync_copy(pidx_hbm.at[pl.ds(base, rows_per_worker)], idx_v)
    pltpu.async_copy(table_hbm.at[idx_v], rows_v, sem).wait()
    pltpu.sync_copy(rows_v, out_hbm.at[1].at[pl.ds(base, rows_per_worker)])

  return gather_kernel(pjpe, jidx, pidx)


def _tc_add_body(x_ref, jp_ref, pp_ref, out_ref, tab_ref):
  @pl.when(pl.program_id(0) == 0)
  def _build_table():
    half = jnp.concatenate([jp_ref[...], pp_ref[...]], axis=-1)
    tab_ref[...] = jnp.concatenate([half] * HEADS, axis=-1)

  out_ref[...] = x_ref[...] + tab_ref[...][None]


def kernel(x, parents, pjpe):
  frame_num, bs, n_joints, d_model = x.shape
  rows = bs * n_joints
  rows_per_worker = -(-rows // NUM_WORKERS)
  rows_per_worker = -(-rows_per_worker // 8) * 8
  rows_pad = rows_per_worker * NUM_WORKERS

  zero_row = pjpe.shape[0] - 1
  r = jnp.arange(rows_pad, dtype=jnp.int32)
  jidx = jnp.where(r < rows, r % n_joints, zero_row)
  pidx = jnp.concatenate([
      parents.reshape(-1).astype(jnp.int32),
      jnp.full((rows_pad - rows,), zero_row, jnp.int32),
  ])

  planes = _sc_gather_rows(pjpe, jidx, pidx, rows_pad, rows_per_worker)
  jp = planes[0, :rows]
  pp = planes[1, :rows]

  xr = x.reshape(frame_num, rows, d_model)
  out = pl.pallas_call(
      _tc_add_body,
      grid=(frame_num,),
      in_specs=[
          pl.BlockSpec((1, rows, d_model), lambda f: (f, 0, 0)),
          pl.BlockSpec((rows, PE_DIM), lambda f: (0, 0)),
          pl.BlockSpec((rows, PE_DIM), lambda f: (0, 0)),
      ],
      out_specs=pl.BlockSpec((1, rows, d_model), lambda f: (f, 0, 0)),
      out_shape=jax.ShapeDtypeStruct((frame_num, rows, d_model), x.dtype),
      scratch_shapes=[pltpu.VMEM((rows, d_model), jnp.float32)],
  )(xr, jp, pp)
  return out.reshape(frame_num, bs, n_joints, d_model)


# speedup vs baseline: 1.3353x; 1.3353x over previous
import functools

import jax
import jax.numpy as jnp
from jax import lax
from jax.experimental import pallas as pl
from jax.experimental.pallas import tpu as pltpu
from jax.experimental.pallas import tpu_sc as plsc

HEADS = 4
PE_DIM = 64
NUM_CORES = 2
NUM_SUBCORES = 16
NUM_WORKERS = NUM_CORES * NUM_SUBCORES


def _sc_gather_rows(pjpe, cidx, total_rows, rows_per_worker):
  mesh = plsc.VectorSubcoreMesh(
      core_axis_name="c",
      subcore_axis_name="s",
      num_cores=NUM_CORES,
      num_subcores=NUM_SUBCORES,
  )

  @functools.partial(
      pl.kernel,
      mesh=mesh,
      compiler_params=pltpu.CompilerParams(use_tc_tiling_on_sc=False),
      out_type=jax.ShapeDtypeStruct((total_rows, PE_DIM), jnp.float32),
      scratch_types=[
          pltpu.VMEM((rows_per_worker,), jnp.int32),
          pltpu.VMEM((rows_per_worker, PE_DIM), jnp.float32),
          pltpu.SemaphoreType.DMA,
      ],
  )
  def gather_kernel(table_hbm, cidx_hbm, out_hbm, idx_v, rows_v, sem):
    wid = lax.axis_index("s") * NUM_CORES + lax.axis_index("c")
    base = wid * rows_per_worker
    pltpu.sync_copy(cidx_hbm.at[pl.ds(base, rows_per_worker)], idx_v)
    pltpu.async_copy(table_hbm.at[idx_v], rows_v, sem).wait()
    pltpu.sync_copy(rows_v, out_hbm.at[pl.ds(base, rows_per_worker)])

  return gather_kernel(pjpe, cidx)


def _tc_add_body(x_ref, jp_ref, pp_ref, out_ref, tab_ref):
  n_joints = x_ref.shape[2]

  @pl.when(pl.program_id(0) == 0)
  def _build_table():
    half = jnp.concatenate(
        [jp_ref[:, :n_joints, :], pp_ref[:, :n_joints, :]], axis=-1
    )
    tab_ref[...] = jnp.concatenate([half] * HEADS, axis=-1)

  out_ref[...] = x_ref[...] + tab_ref[...][None]


def kernel(x, parents, pjpe):
  frame_num, bs, n_joints, d_model = x.shape
  zero_row = pjpe.shape[0] - 1
  j_pad = -(-(n_joints + 1) // 8) * 8
  rows_pad = bs * j_pad
  rows_per_worker = (2 * rows_pad) // NUM_WORKERS
  assert 2 * rows_pad == rows_per_worker * NUM_WORKERS and rows_per_worker % 8 == 0

  r = jnp.arange(rows_pad, dtype=jnp.int32)
  jidx = jnp.minimum(r % j_pad, zero_row)
  pidx = jnp.concatenate(
      [parents.astype(jnp.int32),
       jnp.full((bs, j_pad - n_joints), zero_row, jnp.int32)],
      axis=1,
  ).reshape(-1)
  cidx = jnp.concatenate([jidx, pidx])

  planes = _sc_gather_rows(pjpe, cidx, 2 * rows_pad, rows_per_worker)
  planes = planes.reshape(2, bs, j_pad, PE_DIM)

  out = pl.pallas_call(
      _tc_add_body,
      grid=(frame_num,),
      in_specs=[
          pl.BlockSpec((1, bs, n_joints, d_model), lambda f: (f, 0, 0, 0)),
          pl.BlockSpec((bs, j_pad, PE_DIM), lambda f: (0, 0, 0)),
          pl.BlockSpec((bs, j_pad, PE_DIM), lambda f: (0, 0, 0)),
      ],
      out_specs=pl.BlockSpec((1, bs, n_joints, d_model), lambda f: (f, 0, 0, 0)),
      out_shape=jax.ShapeDtypeStruct(x.shape, x.dtype),
      scratch_shapes=[pltpu.VMEM((bs, n_joints, d_model), jnp.float32)],
  )(x, planes[0], planes[1])
  return out
